# baseline (speedup 1.0000x reference)
import functools

import jax
import jax.numpy as jnp
from jax import lax
from jax.experimental import pallas as pl
from jax.experimental.pallas import tpu as pltpu
from jax.experimental.pallas import tpu_sc as plsc

N = 320000
D = 128
S = 10000
L = 16
NC = 2
NS = 16
NW = NC * NS
SEG_W = 312
SEG_LAST = S - SEG_W * (NW - 1)
OBUF_ROWS = SEG_LAST + 1
R = 160
NBUF = 4
SUB = 64
NVEC = D // L


def _seg_min_call(emb_flat, idx, starts):
    mesh = plsc.VectorSubcoreMesh(
        core_axis_name="c", subcore_axis_name="s",
        num_cores=NC, num_subcores=NS)

    @functools.partial(
        pl.kernel,
        mesh=mesh,
        out_type=jax.ShapeDtypeStruct((S * D,), jnp.float32),
        scratch_types=[
            pltpu.VMEM((48,), jnp.int32),
            [pltpu.VMEM((R * D,), jnp.float32)] * NBUF,
            [pltpu.VMEM((R + L,), jnp.int32)] * NBUF,
            pltpu.VMEM((OBUF_ROWS * D,), jnp.float32),
            pltpu.VMEM((D,), jnp.float32),
            [pltpu.SemaphoreType.DMA] * NBUF,
        ],
    )
    def k(emb_hbm, idx_hbm, starts_hbm, out_hbm,
          starts_v, ebufs, ibufs, obuf, acc_buf, sems):
        wid = lax.axis_index("s") * NC + lax.axis_index("c")
        lo = wid * SEG_W

        hi = lax.select(wid == NW - 1, jnp.int32(S), lo + SEG_W)

        pltpu.sync_copy(starts_hbm, starts_v.at[pl.ds(0, 40)])
        sv = starts_v[pl.ds(wid, L)]
        r0 = jnp.maximum(sv[0] - 1, 0) * SUB
        r1 = sv[1] * SUB

        inf_v = jnp.full((L,), jnp.inf, dtype=jnp.float32)

        def init_body(i, c):
            for j in range(NVEC):
                obuf[pl.ds(i * D + j * L, L)] = inf_v
            return c

        lax.fori_loop(0, OBUF_ROWS, init_body, 0)
        for j in range(NVEC):
            acc_buf[pl.ds(j * L, L)] = inf_v

        def load_acc():
            return tuple(acc_buf[pl.ds(j * L, L)] for j in range(NVEC))

        def store_acc(acc):
            for j in range(NVEC):
                acc_buf[pl.ds(j * L, L)] = acc[j]

        def _copies(b, p):
            return (
                pltpu.make_async_copy(
                    emb_hbm.at[pl.ds(b * R * D, R * D)], ebufs[p], sems[p]),
                pltpu.make_async_copy(
                    idx_hbm.at[pl.ds(b * R, R)],
                    ibufs[p].at[pl.ds(0, R)], sems[p]),
            )

        def _slot(c):
            return lax.select((c >= lo) & (c < hi), c - lo + 1, jnp.int32(0))

        def row_body(b, ebuf, ibuf):
            def body(r, carry):
                cur = carry[-1]
                acc = carry[:-1]
                local = r - b * R
                seg = ibuf[pl.ds(local, L)][0]
                base = _slot(cur) * D
                for j in range(NVEC):
                    obuf[pl.ds(base + j * L, L)] = acc[j]
                pen = lax.select(seg != cur,
                                 jnp.float32(jnp.inf), jnp.float32(0.0))
                ebase = local * D
                new_acc = tuple(
                    jnp.minimum(acc[j] + pen,
                                ebuf[pl.ds(ebase + j * L, L)])
                    for j in range(NVEC)
                )
                return new_acc + (seg,)

            return body

        def grp_body(b, ebuf, ibuf):
            def body(g, cur):
                lbase = g * L - b * R
                iv = ibuf[pl.ds(lbase, L)]
                c = cur
                acc = list(load_acc())
                for kk in range(L):
                    seg = iv[kk]
                    basew = _slot(c) * D
                    for j in range(NVEC):
                        obuf[pl.ds(basew + j * L, L)] = acc[j]
                    pen = lax.select(seg != c,
                                     jnp.float32(jnp.inf), jnp.float32(0.0))
                    ebase = (lbase + kk) * D
                    for j in range(NVEC):
                        acc[j] = jnp.minimum(acc[j] + pen,
                                             ebuf[pl.ds(ebase + j * L, L)])
                    c = seg
                store_acc(acc)
                return iv[L - 1]

            return body

        b0 = r0 // R
        b1 = lax.select(r1 > r0, (r1 + R - 1) // R, r0 // R)

        for i in range(NBUF - 1):
            @pl.when(b0 + i < b1)
            def _(i=i):
                for cp in _copies(b0 + i, i):
                    cp.start()

        def pair_body(g, cur):
            for p in range(NBUF):
                b = b0 + NBUF * g + p

                @pl.when(b + NBUF - 1 < b1)
                def _():
                    for cp in _copies(b + NBUF - 1, (p - 1) % NBUF):
                        cp.start()

                @pl.when(b < b1)
                def _():
                    for cp in _copies(b, p):
                        cp.wait()

                rs = jnp.maximum(r0, b * R)
                re = jnp.minimum(r1, (b + 1) * R)
                afs = ((rs + L - 1) // L) * L
                afe = (re // L) * L
                hc = lax.fori_loop(rs, jnp.minimum(afs, re),
                                   row_body(b, ebufs[p], ibufs[p]),
                                   load_acc() + (cur,))
                store_acc(hc[:-1])
                cur = lax.fori_loop(afs // L, jnp.maximum(afe, afs) // L,
                                    grp_body(b, ebufs[p], ibufs[p]), hc[-1])
                tc = lax.fori_loop(jnp.maximum(afe, jnp.minimum(afs, re)),
                                   re,
                                   row_body(b, ebufs[p], ibufs[p]),
                                   load_acc() + (cur,))
                store_acc(tc[:-1])
                cur = tc[-1]
            return cur

        npairs = (b1 - b0 + NBUF - 1) // NBUF
        cur = lax.fori_loop(0, npairs, pair_body, lo - 1)

        acc = load_acc()
        base = _slot(cur) * D
        for j in range(NVEC):
            obuf[pl.ds(base + j * L, L)] = acc[j]

        @pl.when(wid < NW - 1)
        def _():
            pltpu.sync_copy(obuf.at[pl.ds(D, SEG_W * D)],
                            out_hbm.at[pl.ds(lo * D, SEG_W * D)])

        @pl.when(wid == NW - 1)
        def _():
            pltpu.sync_copy(obuf.at[pl.ds(D, SEG_LAST * D)],
                            out_hbm.at[pl.ds(lo * D, SEG_LAST * D)])

    return k(emb_flat, idx, starts)


def kernel(embeddings, idx, dim_size):
    del dim_size
    ar = jnp.arange(NW + 1, dtype=jnp.int32)
    bounds = jnp.where(ar == NW, jnp.int32(S), ar * SEG_W)
    sub = idx[::SUB]
    cs = jnp.sum(sub[:, None] < bounds[None, :], axis=0, dtype=jnp.int32)
    starts = jnp.concatenate(
        [cs, jnp.full((40 - NW - 1,), N // SUB, dtype=jnp.int32)])
    out = _seg_min_call(embeddings.reshape(N * D), idx, starts)
    return out.reshape(S, D)


# speedup vs baseline: 1.0525x; 1.0525x over previous
import functools

import jax
import jax.numpy as jnp
from jax import lax
from jax.experimental import pallas as pl
from jax.experimental.pallas import tpu as pltpu
from jax.experimental.pallas import tpu_sc as plsc

N = 320000
D = 128
S = 10000
L = 16
NC = 2
NS = 16
NW = NC * NS
SEG_W = 312
SEG_LAST = S - SEG_W * (NW - 1)
OBUF_ROWS = SEG_LAST + 1
R = 320
NBUF = 2
SUB = 64
NVEC = D // L


def _seg_min_call(emb_flat, idx, starts):
    mesh = plsc.VectorSubcoreMesh(
        core_axis_name="c", subcore_axis_name="s",
        num_cores=NC, num_subcores=NS)

    @functools.partial(
        pl.kernel,
        mesh=mesh,
        out_type=jax.ShapeDtypeStruct((S * D,), jnp.float32),
        scratch_types=[
            pltpu.VMEM((48,), jnp.int32),
            [pltpu.VMEM((R * D,), jnp.float32)] * NBUF,
            [pltpu.VMEM((R + L,), jnp.int32)] * NBUF,
            pltpu.VMEM((OBUF_ROWS * D,), jnp.float32),
            pltpu.VMEM((D,), jnp.float32),
            [pltpu.SemaphoreType.DMA] * NBUF,
        ],
    )
    def k(emb_hbm, idx_hbm, starts_hbm, out_hbm,
          starts_v, ebufs, ibufs, obuf, acc_buf, sems):
        wid = lax.axis_index("s") * NC + lax.axis_index("c")
        lo = wid * SEG_W

        hi = lax.select(wid == NW - 1, jnp.int32(S), lo + SEG_W)

        pltpu.sync_copy(starts_hbm, starts_v.at[pl.ds(0, 40)])
        sv = starts_v[pl.ds(wid, L)]
        r0 = jnp.maximum(sv[0] - 1, 0) * SUB
        r1 = sv[1] * SUB

        inf_v = jnp.full((L,), jnp.inf, dtype=jnp.float32)

        def init_body(i, c):
            for j in range(NVEC):
                obuf[pl.ds(i * D + j * L, L)] = inf_v
            return c

        lax.fori_loop(0, OBUF_ROWS, init_body, 0)
        for j in range(NVEC):
            acc_buf[pl.ds(j * L, L)] = inf_v

        def load_acc():
            return tuple(acc_buf[pl.ds(j * L, L)] for j in range(NVEC))

        def store_acc(acc):
            for j in range(NVEC):
                acc_buf[pl.ds(j * L, L)] = acc[j]

        def _copies(b, p):
            return (
                pltpu.make_async_copy(
                    emb_hbm.at[pl.ds(b * R * D, R * D)], ebufs[p], sems[p]),
                pltpu.make_async_copy(
                    idx_hbm.at[pl.ds(b * R, R)],
                    ibufs[p].at[pl.ds(0, R)], sems[p]),
            )

        def _slot(c):
            return lax.select((c >= lo) & (c < hi), c - lo + 1, jnp.int32(0))

        def row_body(b, ebuf, ibuf):
            def body(r, carry):
                cur = carry[-1]
                acc = carry[:-1]
                local = r - b * R
                seg = ibuf[pl.ds(local, L)][0]
                base = _slot(cur) * D
                for j in range(NVEC):
                    obuf[pl.ds(base + j * L, L)] = acc[j]
                pen = lax.select(seg != cur,
                                 jnp.float32(jnp.inf), jnp.float32(0.0))
                ebase = local * D
                new_acc = tuple(
                    jnp.minimum(acc[j] + pen,
                                ebuf[pl.ds(ebase + j * L, L)])
                    for j in range(NVEC)
                )
                return new_acc + (seg,)

            return body

        def grp_body(b, ebuf, ibuf):
            def body(g, cur):
                lbase = g * L - b * R
                iv = ibuf[pl.ds(lbase, L)]
                c = cur
                acc = list(load_acc())
                for kk in range(L):
                    seg = iv[kk]
                    basew = _slot(c) * D
                    for j in range(NVEC):
                        obuf[pl.ds(basew + j * L, L)] = acc[j]
                    pen = lax.select(seg != c,
                                     jnp.float32(jnp.inf), jnp.float32(0.0))
                    ebase = (lbase + kk) * D
                    for j in range(NVEC):
                        acc[j] = jnp.minimum(acc[j] + pen,
                                             ebuf[pl.ds(ebase + j * L, L)])
                    c = seg
                store_acc(acc)
                return iv[L - 1]

            return body

        b0 = r0 // R
        b1 = lax.select(r1 > r0, (r1 + R - 1) // R, r0 // R)

        for i in range(NBUF - 1):
            @pl.when(b0 + i < b1)
            def _(i=i):
                for cp in _copies(b0 + i, i):
                    cp.start()

        def pair_body(g, cur):
            for p in range(NBUF):
                b = b0 + NBUF * g + p

                @pl.when(b + NBUF - 1 < b1)
                def _():
                    for cp in _copies(b + NBUF - 1, (p - 1) % NBUF):
                        cp.start()

                @pl.when(b < b1)
                def _():
                    for cp in _copies(b, p):
                        cp.wait()

                rs = jnp.maximum(r0, b * R)
                re = jnp.minimum(r1, (b + 1) * R)
                afs = ((rs + L - 1) // L) * L
                afe = (re // L) * L
                hc = lax.fori_loop(rs, jnp.minimum(afs, re),
                                   row_body(b, ebufs[p], ibufs[p]),
                                   load_acc() + (cur,))
                store_acc(hc[:-1])
                cur = lax.fori_loop(afs // L, jnp.maximum(afe, afs) // L,
                                    grp_body(b, ebufs[p], ibufs[p]), hc[-1])
                tc = lax.fori_loop(jnp.maximum(afe, jnp.minimum(afs, re)),
                                   re,
                                   row_body(b, ebufs[p], ibufs[p]),
                                   load_acc() + (cur,))
                store_acc(tc[:-1])
                cur = tc[-1]
            return cur

        npairs = (b1 - b0 + NBUF - 1) // NBUF
        cur = lax.fori_loop(0, npairs, pair_body, lo - 1)

        acc = load_acc()
        base = _slot(cur) * D
        for j in range(NVEC):
            obuf[pl.ds(base + j * L, L)] = acc[j]

        @pl.when(wid < NW - 1)
        def _():
            pltpu.sync_copy(obuf.at[pl.ds(D, SEG_W * D)],
                            out_hbm.at[pl.ds(lo * D, SEG_W * D)])

        @pl.when(wid == NW - 1)
        def _():
            pltpu.sync_copy(obuf.at[pl.ds(D, SEG_LAST * D)],
                            out_hbm.at[pl.ds(lo * D, SEG_LAST * D)])

    return k(emb_flat, idx, starts)


def kernel(embeddings, idx, dim_size):
    del dim_size
    ar = jnp.arange(NW + 1, dtype=jnp.int32)
    bounds = jnp.where(ar == NW, jnp.int32(S), ar * SEG_W)
    sub = idx[::SUB]
    cs = jnp.sum(sub[:, None] < bounds[None, :], axis=0, dtype=jnp.int32)
    starts = jnp.concatenate(
        [cs, jnp.full((40 - NW - 1,), N // SUB, dtype=jnp.int32)])
    out = _seg_min_call(embeddings.reshape(N * D), idx, starts)
    return out.reshape(S, D)


# speedup vs baseline: 1.0547x; 1.0021x over previous
import functools

import jax
import jax.numpy as jnp
from jax import lax
from jax.experimental import pallas as pl
from jax.experimental.pallas import tpu as pltpu
from jax.experimental.pallas import tpu_sc as plsc

N = 320000
D = 128
S = 10000
L = 16
NC = 2
NS = 16
NW = NC * NS
SEG_W = 312
SEG_LAST = S - SEG_W * (NW - 1)
OBUF_ROWS = SEG_LAST + 1
R = 320
NBUF = 2
SUB = 64
NVEC = D // L


def _seg_min_call(emb_flat, idx, starts):
    mesh = plsc.VectorSubcoreMesh(
        core_axis_name="c", subcore_axis_name="s",
        num_cores=NC, num_subcores=NS)

    @functools.partial(
        pl.kernel,
        mesh=mesh,
        out_type=jax.ShapeDtypeStruct((S * D,), jnp.float32),
        scratch_types=[
            pltpu.VMEM((48,), jnp.int32),
            [pltpu.VMEM((R * D,), jnp.float32)] * NBUF,
            [pltpu.VMEM((R + L,), jnp.int32)] * NBUF,
            pltpu.VMEM((OBUF_ROWS * D,), jnp.float32),
            pltpu.VMEM((D,), jnp.float32),
            [pltpu.SemaphoreType.DMA] * NBUF,
        ],
    )
    def k(emb_hbm, idx_hbm, starts_hbm, out_hbm,
          starts_v, ebufs, ibufs, obuf, acc_buf, sems):
        wid = lax.axis_index("s") * NC + lax.axis_index("c")
        lo = wid * SEG_W

        hi = lax.select(wid == NW - 1, jnp.int32(S), lo + SEG_W)

        scp = pltpu.make_async_copy(starts_hbm, starts_v.at[pl.ds(0, 40)],
                                    sems[0])
        scp.start()

        inf_v = jnp.full((L,), jnp.inf, dtype=jnp.float32)

        def init_body(i, c):
            for j in range(NVEC):
                obuf[pl.ds(i * D + j * L, L)] = inf_v
            return c

        lax.fori_loop(0, OBUF_ROWS, init_body, 0)
        for j in range(NVEC):
            acc_buf[pl.ds(j * L, L)] = inf_v

        scp.wait()
        sv = starts_v[pl.ds(wid, L)]
        r0 = jnp.maximum(sv[0] - 1, 0) * SUB
        r1 = sv[1] * SUB

        def load_acc():
            return tuple(acc_buf[pl.ds(j * L, L)] for j in range(NVEC))

        def store_acc(acc):
            for j in range(NVEC):
                acc_buf[pl.ds(j * L, L)] = acc[j]

        def _copies(b, p):
            return (
                pltpu.make_async_copy(
                    emb_hbm.at[pl.ds(b * R * D, R * D)], ebufs[p], sems[p]),
                pltpu.make_async_copy(
                    idx_hbm.at[pl.ds(b * R, R)],
                    ibufs[p].at[pl.ds(0, R)], sems[p]),
            )

        def _slot(c):
            return lax.select((c >= lo) & (c < hi), c - lo + 1, jnp.int32(0))

        def row_body(b, ebuf, ibuf):
            def body(r, carry):
                cur = carry[-1]
                acc = carry[:-1]
                local = r - b * R
                seg = ibuf[pl.ds(local, L)][0]
                base = _slot(cur) * D
                for j in range(NVEC):
                    obuf[pl.ds(base + j * L, L)] = acc[j]
                pen = lax.select(seg != cur,
                                 jnp.float32(jnp.inf), jnp.float32(0.0))
                ebase = local * D
                new_acc = tuple(
                    jnp.minimum(acc[j] + pen,
                                ebuf[pl.ds(ebase + j * L, L)])
                    for j in range(NVEC)
                )
                return new_acc + (seg,)

            return body

        def grp_body(b, ebuf, ibuf):
            def body(g, cur):
                lbase = g * L - b * R
                iv = ibuf[pl.ds(lbase, L)]
                c = cur
                acc = list(load_acc())
                for kk in range(L):
                    seg = iv[kk]
                    basew = _slot(c) * D
                    for j in range(NVEC):
                        obuf[pl.ds(basew + j * L, L)] = acc[j]
                    pen = lax.select(seg != c,
                                     jnp.float32(jnp.inf), jnp.float32(0.0))
                    ebase = (lbase + kk) * D
                    for j in range(NVEC):
                        acc[j] = jnp.minimum(acc[j] + pen,
                                             ebuf[pl.ds(ebase + j * L, L)])
                    c = seg
                store_acc(acc)
                return iv[L - 1]

            return body

        b0 = r0 // R
        b1 = lax.select(r1 > r0, (r1 + R - 1) // R, r0 // R)

        for i in range(NBUF - 1):
            @pl.when(b0 + i < b1)
            def _(i=i):
                for cp in _copies(b0 + i, i):
                    cp.start()

        def pair_body(g, cur):
            for p in range(NBUF):
                b = b0 + NBUF * g + p

                @pl.when(b + NBUF - 1 < b1)
                def _():
                    for cp in _copies(b + NBUF - 1, (p - 1) % NBUF):
                        cp.start()

                @pl.when(b < b1)
                def _():
                    for cp in _copies(b, p):
                        cp.wait()

                rs = jnp.maximum(r0, b * R)
                re = jnp.minimum(r1, (b + 1) * R)
                afs = ((rs + L - 1) // L) * L
                afe = (re // L) * L
                hc = lax.fori_loop(rs, jnp.minimum(afs, re),
                                   row_body(b, ebufs[p], ibufs[p]),
                                   load_acc() + (cur,))
                store_acc(hc[:-1])
                cur = lax.fori_loop(afs // L, jnp.maximum(afe, afs) // L,
                                    grp_body(b, ebufs[p], ibufs[p]), hc[-1])
                tc = lax.fori_loop(jnp.maximum(afe, jnp.minimum(afs, re)),
                                   re,
                                   row_body(b, ebufs[p], ibufs[p]),
                                   load_acc() + (cur,))
                store_acc(tc[:-1])
                cur = tc[-1]
            return cur

        npairs = (b1 - b0 + NBUF - 1) // NBUF
        cur = lax.fori_loop(0, npairs, pair_body, lo - 1)

        acc = load_acc()
        base = _slot(cur) * D
        for j in range(NVEC):
            obuf[pl.ds(base + j * L, L)] = acc[j]

        @pl.when(wid < NW - 1)
        def _():
            pltpu.sync_copy(obuf.at[pl.ds(D, SEG_W * D)],
                            out_hbm.at[pl.ds(lo * D, SEG_W * D)])

        @pl.when(wid == NW - 1)
        def _():
            pltpu.sync_copy(obuf.at[pl.ds(D, SEG_LAST * D)],
                            out_hbm.at[pl.ds(lo * D, SEG_LAST * D)])

    return k(emb_flat, idx, starts)


def kernel(embeddings, idx, dim_size):
    del dim_size
    ar = jnp.arange(NW + 1, dtype=jnp.int32)
    bounds = jnp.where(ar == NW, jnp.int32(S), ar * SEG_W)
    sub = idx[::SUB]
    cs = jnp.sum(sub[:, None] < bounds[None, :], axis=0, dtype=jnp.int32)
    starts = jnp.concatenate(
        [cs, jnp.full((40 - NW - 1,), N // SUB, dtype=jnp.int32)])
    out = _seg_min_call(embeddings.reshape(N * D), idx, starts)
    return out.reshape(S, D)
